# baseline (speedup 1.0000x reference)
import functools

import jax
import jax.numpy as jnp
from jax import lax
from jax.experimental import pallas as pl
from jax.experimental.pallas import tpu as pltpu
from jax.experimental.pallas import tpu_sc as plsc

NE = 8
DM = 1024
DF = 2048
T = 2048
BT = 128
NT = (T * 2 + NE * (BT - 1) + BT - 1) // BT
P = NT * BT
NC, NS = 2, 16
NW = NC * NS
TPW = T // NW
KB = 256


def _route_body(x_ref, wr_ref, dsta_ref, dstb_ref, wa_ref, wb_ref, te_ref):
    x = x_ref[...]
    logits = jnp.dot(x, wr_ref[...], preferred_element_type=jnp.float32)
    m = jnp.max(logits, axis=1, keepdims=True)
    ex = jnp.exp(logits - m)
    probs = ex / jnp.sum(ex, axis=1, keepdims=True)

    eio = lax.broadcasted_iota(jnp.int32, (T, NE), 1)
    m1 = jnp.max(probs, axis=1, keepdims=True)
    i1 = jnp.min(jnp.where(probs == m1, eio, NE), axis=1, keepdims=True)
    p2 = jnp.where(eio == i1, -1.0, probs)
    m2 = jnp.max(p2, axis=1, keepdims=True)
    i2 = jnp.min(jnp.where(p2 == m2, eio, NE), axis=1, keepdims=True)
    s = m1 + m2
    wa_ref[...] = m1 / s
    wb_ref[...] = m2 / s

    oha = (eio == i1).astype(jnp.float32)
    ohb = (eio == i2).astype(jnp.float32)
    ind = oha + ohb

    pos = jnp.zeros((T, NE), jnp.float32)
    rio = lax.broadcasted_iota(jnp.int32, (T, KB), 0)
    cio = lax.broadcasted_iota(jnp.int32, (T, KB), 1)
    indb = ind.astype(jnp.bfloat16)
    for kb in range(T // KB):
        tri = (rio > cio + kb * KB).astype(jnp.bfloat16)
        pos = pos + jnp.dot(tri, indb[kb * KB:(kb + 1) * KB, :],
                            preferred_element_type=jnp.float32)

    counts = jnp.sum(ind, axis=0, keepdims=True)
    tiles = jnp.floor((counts + (BT - 1)) * (1.0 / BT))
    ii = lax.broadcasted_iota(jnp.int32, (NE, NE), 0)
    jj = lax.broadcasted_iota(jnp.int32, (NE, NE), 1)
    excl = (ii < jj).astype(jnp.float32)
    start_tiles = jnp.dot(tiles, excl, preferred_element_type=jnp.float32)

    dest = start_tiles * BT + pos
    dsta_ref[...] = jnp.sum(dest * oha, axis=1).astype(jnp.int32)
    dstb_ref[...] = jnp.sum(dest * ohb, axis=1).astype(jnp.int32)

    tio = lax.broadcasted_iota(jnp.int32, (NT, NE), 0)
    st_i = start_tiles.astype(jnp.int32)
    te_ref[...] = jnp.sum((tio >= st_i).astype(jnp.int32), axis=1) - 1


_route = pl.pallas_call(
    _route_body,
    out_shape=[
        jax.ShapeDtypeStruct((T,), jnp.int32),
        jax.ShapeDtypeStruct((T,), jnp.int32),
        jax.ShapeDtypeStruct((T, 1), jnp.float32),
        jax.ShapeDtypeStruct((T, 1), jnp.float32),
        jax.ShapeDtypeStruct((NT,), jnp.int32),
    ],
)


def _dispatch_body(x_hbm, dsta_hbm, dstb_hbm, xs_hbm, idx_v, rows_v, sem):
    wid = lax.axis_index("s") * NC + lax.axis_index("c")
    base = wid * TPW
    pltpu.sync_copy(x_hbm.at[pl.ds(base, TPW)], rows_v)
    pltpu.sync_copy(dsta_hbm.at[wid], idx_v)
    pltpu.async_copy(rows_v, xs_hbm.at[idx_v], sem).wait()
    pltpu.sync_copy(dstb_hbm.at[wid], idx_v)
    pltpu.async_copy(rows_v, xs_hbm.at[idx_v], sem).wait()


def _gmm_body(te_ref, xs_ref, w1_ref, w2_ref, out_ref, w1b_s, w2b_s):
    j = pl.program_id(0)
    e = te_ref[j]
    prev_e = te_ref[jnp.maximum(j - 1, 0)]

    @pl.when((j == 0) | (e != prev_e))
    def _cast_weights():
        w1b_s[...] = w1_ref[0].astype(jnp.bfloat16)
        w2b_s[...] = w2_ref[0].astype(jnp.bfloat16)

    h = jnp.dot(xs_ref[...].astype(jnp.bfloat16), w1b_s[...],
                preferred_element_type=jnp.float32)
    h = h * (1.0 / (1.0 + jnp.exp(-h)))
    out_ref[...] = jnp.dot(h.astype(jnp.bfloat16), w2b_s[...],
                           preferred_element_type=jnp.float32)


_gmm = pl.pallas_call(
    _gmm_body,
    grid_spec=pltpu.PrefetchScalarGridSpec(
        num_scalar_prefetch=1,
        grid=(NT,),
        in_specs=[
            pl.BlockSpec((BT, DM), lambda j, te: (j, 0)),
            pl.BlockSpec((1, DM, DF), lambda j, te: (te[j], 0, 0)),
            pl.BlockSpec((1, DF, DM), lambda j, te: (te[j], 0, 0)),
        ],
        out_specs=pl.BlockSpec((BT, DM), lambda j, te: (j, 0)),
        scratch_shapes=[
            pltpu.VMEM((DM, DF), jnp.bfloat16),
            pltpu.VMEM((DF, DM), jnp.bfloat16),
        ],
    ),
    out_shape=jax.ShapeDtypeStruct((P, DM), jnp.float32),
)


def _gather2_body(h2_hbm, dsta_hbm, dstb_hbm, ga_hbm, gb_hbm, idx_v, buf_v, sem):
    wid = lax.axis_index("s") * NC + lax.axis_index("c")
    base = wid * TPW
    pltpu.sync_copy(dsta_hbm.at[wid], idx_v)
    pltpu.async_copy(h2_hbm.at[idx_v], buf_v, sem).wait()
    pltpu.sync_copy(buf_v, ga_hbm.at[pl.ds(base, TPW)])
    pltpu.sync_copy(dstb_hbm.at[wid], idx_v)
    pltpu.async_copy(h2_hbm.at[idx_v], buf_v, sem).wait()
    pltpu.sync_copy(buf_v, gb_hbm.at[pl.ds(base, TPW)])


def _combine_body(ga_ref, gb_ref, wa_ref, wb_ref, out_ref):
    out_ref[...] = wa_ref[...] * ga_ref[...] + wb_ref[...] * gb_ref[...]


BC = 256
_combine = pl.pallas_call(
    _combine_body,
    grid=(T // BC,),
    in_specs=[
        pl.BlockSpec((BC, DM), lambda i: (i, 0)),
        pl.BlockSpec((BC, DM), lambda i: (i, 0)),
        pl.BlockSpec((BC, 1), lambda i: (i, 0)),
        pl.BlockSpec((BC, 1), lambda i: (i, 0)),
    ],
    out_specs=pl.BlockSpec((BC, DM), lambda i: (i, 0)),
    out_shape=jax.ShapeDtypeStruct((T, DM), jnp.float32),
)


@functools.lru_cache(maxsize=1)
def _sc_kernels():
    mesh = plsc.VectorSubcoreMesh(
        core_axis_name="c", subcore_axis_name="s",
        num_cores=NC, num_subcores=NS)
    dispatch = pl.kernel(
        _dispatch_body,
        mesh=mesh,
        out_type=jax.ShapeDtypeStruct((P, DM), jnp.float32),
        scratch_types=[
            pltpu.VMEM((TPW,), jnp.int32),
            pltpu.VMEM((TPW, DM), jnp.float32),
            pltpu.SemaphoreType.DMA,
        ],
    )
    gather2 = pl.kernel(
        _gather2_body,
        mesh=mesh,
        out_type=[
            jax.ShapeDtypeStruct((T, DM), jnp.float32),
            jax.ShapeDtypeStruct((T, DM), jnp.float32),
        ],
        scratch_types=[
            pltpu.VMEM((TPW,), jnp.int32),
            pltpu.VMEM((TPW, DM), jnp.float32),
            pltpu.SemaphoreType.DMA,
        ],
    )
    return dispatch, gather2


def kernel(x, w_router, w1, w2):
    dispatch, gather2 = _sc_kernels()
    dsta, dstb, wa, wb, te = _route(x, w_router)
    da = dsta.reshape(NW, TPW)
    db = dstb.reshape(NW, TPW)
    xs = dispatch(x, da, db)
    h2 = _gmm(te, xs, w1, w2)
    ga, gb = gather2(h2, da, db)
    return _combine(ga, gb, wa, wb)


# speedup vs baseline: 1.0919x; 1.0919x over previous
import functools

import jax
import jax.numpy as jnp
from jax import lax
from jax.experimental import pallas as pl
from jax.experimental.pallas import tpu as pltpu
from jax.experimental.pallas import tpu_sc as plsc

NE = 8
DM = 1024
DF = 2048
T = 2048
BT = 128
NT = (T * 2 + NE * (BT - 1) + BT - 1) // BT
P = NT * BT
NC, NS = 2, 16
NW = NC * NS
TPW = T // NW
KB = 256


def _route_body(x_ref, wr_ref, dsta_ref, dstb_ref, wa_ref, wb_ref, te_ref):
    x = x_ref[...]
    logits = jnp.dot(x, wr_ref[...], preferred_element_type=jnp.float32)
    m = jnp.max(logits, axis=1, keepdims=True)
    ex = jnp.exp(logits - m)
    probs = ex / jnp.sum(ex, axis=1, keepdims=True)

    eio = lax.broadcasted_iota(jnp.int32, (T, NE), 1)
    m1 = jnp.max(probs, axis=1, keepdims=True)
    i1 = jnp.min(jnp.where(probs == m1, eio, NE), axis=1, keepdims=True)
    p2 = jnp.where(eio == i1, -1.0, probs)
    m2 = jnp.max(p2, axis=1, keepdims=True)
    i2 = jnp.min(jnp.where(p2 == m2, eio, NE), axis=1, keepdims=True)
    s = m1 + m2
    wa_ref[...] = m1 / s
    wb_ref[...] = m2 / s

    oha = (eio == i1).astype(jnp.float32)
    ohb = (eio == i2).astype(jnp.float32)
    ind = oha + ohb

    pos = jnp.zeros((T, NE), jnp.float32)
    rio = lax.broadcasted_iota(jnp.int32, (T, KB), 0)
    cio = lax.broadcasted_iota(jnp.int32, (T, KB), 1)
    indb = ind.astype(jnp.bfloat16)
    for kb in range(T // KB):
        tri = (rio > cio + kb * KB).astype(jnp.bfloat16)
        pos = pos + jnp.dot(tri, indb[kb * KB:(kb + 1) * KB, :],
                            preferred_element_type=jnp.float32)

    counts = jnp.sum(ind, axis=0, keepdims=True)
    tiles = jnp.floor((counts + (BT - 1)) * (1.0 / BT))
    ii = lax.broadcasted_iota(jnp.int32, (NE, NE), 0)
    jj = lax.broadcasted_iota(jnp.int32, (NE, NE), 1)
    excl = (ii < jj).astype(jnp.float32)
    start_tiles = jnp.dot(tiles, excl, preferred_element_type=jnp.float32)

    dest = start_tiles * BT + pos
    dsta_ref[...] = jnp.sum(dest * oha, axis=1).astype(jnp.int32)
    dstb_ref[...] = jnp.sum(dest * ohb, axis=1).astype(jnp.int32)

    tio = lax.broadcasted_iota(jnp.int32, (NT, NE), 0)
    eio2 = lax.broadcasted_iota(jnp.int32, (NT, NE), 1)
    st_i = start_tiles.astype(jnp.int32)
    nonempty = counts > 0.0
    started = tio >= st_i
    tev = jnp.sum(started.astype(jnp.int32), axis=1) - 1
    run = jnp.sum((started & nonempty).astype(jnp.int32), axis=1) - 1
    par = jnp.bitwise_and(run, 1)
    first = jnp.max(((tio == st_i) & nonempty).astype(jnp.int32), axis=1)
    nxte = jnp.min(jnp.where((tio < st_i) & nonempty, eio2, NE), axis=1)
    hasnx = (nxte < NE).astype(jnp.int32)
    nxte = jnp.minimum(nxte, NE - 1)
    te_ref[...] = jnp.concatenate(
        [tev[None, :], par[None, :], first[None, :],
         nxte[None, :], hasnx[None, :]], axis=0)


_route = pl.pallas_call(
    _route_body,
    out_shape=[
        jax.ShapeDtypeStruct((T,), jnp.int32),
        jax.ShapeDtypeStruct((T,), jnp.int32),
        jax.ShapeDtypeStruct((T, 1), jnp.float32),
        jax.ShapeDtypeStruct((T, 1), jnp.float32),
        jax.ShapeDtypeStruct((5, NT), jnp.int32),
    ],
)


def _dispatch_body(x_hbm, dsta_hbm, dstb_hbm, xs_hbm, idx_v, rows_v, sem):
    wid = lax.axis_index("s") * NC + lax.axis_index("c")
    base = wid * TPW
    pltpu.sync_copy(x_hbm.at[pl.ds(base, TPW)], rows_v)
    pltpu.sync_copy(dsta_hbm.at[wid], idx_v)
    pltpu.async_copy(rows_v, xs_hbm.at[idx_v], sem).wait()
    pltpu.sync_copy(dstb_hbm.at[wid], idx_v)
    pltpu.async_copy(rows_v, xs_hbm.at[idx_v], sem).wait()


def _gmm_body(meta_ref, xs_ref, w1_hbm, w2_hbm, out_ref, w1s, w2s, sems):
    j = pl.program_id(0)
    e = meta_ref[0, j]
    par = meta_ref[1, j]
    first = meta_ref[2, j]
    nxte = meta_ref[3, j]
    hasnx = meta_ref[4, j]

    @pl.when(j == 0)
    def _start_first_run():
        pltpu.make_async_copy(w1_hbm.at[e], w1s.at[0], sems.at[0]).start()
        pltpu.make_async_copy(w2_hbm.at[e], w2s.at[0], sems.at[0]).start()

    @pl.when(first == 1)
    def _run_boundary():
        pltpu.make_async_copy(w1_hbm.at[e], w1s.at[par], sems.at[par]).wait()
        pltpu.make_async_copy(w2_hbm.at[e], w2s.at[par], sems.at[par]).wait()

        @pl.when(hasnx == 1)
        def _prefetch_next_run():
            pltpu.make_async_copy(
                w1_hbm.at[nxte], w1s.at[1 - par], sems.at[1 - par]).start()
            pltpu.make_async_copy(
                w2_hbm.at[nxte], w2s.at[1 - par], sems.at[1 - par]).start()

    h = jnp.dot(xs_ref[...], w1s[par], preferred_element_type=jnp.float32)
    h = h * (1.0 / (1.0 + jnp.exp(-h)))
    out_ref[...] = jnp.dot(h, w2s[par], preferred_element_type=jnp.float32)


_gmm = pl.pallas_call(
    _gmm_body,
    grid_spec=pltpu.PrefetchScalarGridSpec(
        num_scalar_prefetch=1,
        grid=(NT,),
        in_specs=[
            pl.BlockSpec((BT, DM), lambda j, meta: (j, 0)),
            pl.BlockSpec(memory_space=pl.ANY),
            pl.BlockSpec(memory_space=pl.ANY),
        ],
        out_specs=pl.BlockSpec((BT, DM), lambda j, meta: (j, 0)),
        scratch_shapes=[
            pltpu.VMEM((2, DM, DF), jnp.float32),
            pltpu.VMEM((2, DF, DM), jnp.float32),
            pltpu.SemaphoreType.DMA((2,)),
        ],
    ),
    out_shape=jax.ShapeDtypeStruct((P, DM), jnp.float32),
)


def _gather2_body(h2_hbm, dsta_hbm, dstb_hbm, ga_hbm, gb_hbm, idx_v, buf_v, sem):
    wid = lax.axis_index("s") * NC + lax.axis_index("c")
    base = wid * TPW
    pltpu.sync_copy(dsta_hbm.at[wid], idx_v)
    pltpu.async_copy(h2_hbm.at[idx_v], buf_v, sem).wait()
    pltpu.sync_copy(buf_v, ga_hbm.at[pl.ds(base, TPW)])
    pltpu.sync_copy(dstb_hbm.at[wid], idx_v)
    pltpu.async_copy(h2_hbm.at[idx_v], buf_v, sem).wait()
    pltpu.sync_copy(buf_v, gb_hbm.at[pl.ds(base, TPW)])


def _combine_body(ga_ref, gb_ref, wa_ref, wb_ref, out_ref):
    out_ref[...] = wa_ref[...] * ga_ref[...] + wb_ref[...] * gb_ref[...]


BC = 256
_combine = pl.pallas_call(
    _combine_body,
    grid=(T // BC,),
    in_specs=[
        pl.BlockSpec((BC, DM), lambda i: (i, 0)),
        pl.BlockSpec((BC, DM), lambda i: (i, 0)),
        pl.BlockSpec((BC, 1), lambda i: (i, 0)),
        pl.BlockSpec((BC, 1), lambda i: (i, 0)),
    ],
    out_specs=pl.BlockSpec((BC, DM), lambda i: (i, 0)),
    out_shape=jax.ShapeDtypeStruct((T, DM), jnp.float32),
)


@functools.lru_cache(maxsize=1)
def _sc_kernels():
    mesh = plsc.VectorSubcoreMesh(
        core_axis_name="c", subcore_axis_name="s",
        num_cores=NC, num_subcores=NS)
    dispatch = pl.kernel(
        _dispatch_body,
        mesh=mesh,
        out_type=jax.ShapeDtypeStruct((P, DM), jnp.float32),
        scratch_types=[
            pltpu.VMEM((TPW,), jnp.int32),
            pltpu.VMEM((TPW, DM), jnp.float32),
            pltpu.SemaphoreType.DMA,
        ],
    )
    gather2 = pl.kernel(
        _gather2_body,
        mesh=mesh,
        out_type=[
            jax.ShapeDtypeStruct((T, DM), jnp.float32),
            jax.ShapeDtypeStruct((T, DM), jnp.float32),
        ],
        scratch_types=[
            pltpu.VMEM((TPW,), jnp.int32),
            pltpu.VMEM((TPW, DM), jnp.float32),
            pltpu.SemaphoreType.DMA,
        ],
    )
    return dispatch, gather2


def kernel(x, w_router, w1, w2):
    dispatch, gather2 = _sc_kernels()
    dsta, dstb, wa, wb, meta = _route(x, w_router)
    da = dsta.reshape(NW, TPW)
    db = dstb.reshape(NW, TPW)
    xs = dispatch(x, da, db)
    h2 = _gmm(meta, xs, w1, w2)
    ga, gb = gather2(h2, da, db)
    return _combine(ga, gb, wa, wb)


# speedup vs baseline: 1.1179x; 1.0238x over previous
import functools

import jax
import jax.numpy as jnp
from jax import lax
from jax.experimental import pallas as pl
from jax.experimental.pallas import tpu as pltpu
from jax.experimental.pallas import tpu_sc as plsc

NE = 8
DM = 1024
DF = 2048
T = 2048
BT = 128
NT = (T * 2 + NE * (BT - 1) + BT - 1) // BT
P = NT * BT
NC, NS = 2, 16
NW = NC * NS
TPW = T // NW
KB = 256


def _route_body(x_ref, wr_ref, dsta_ref, dstb_ref, wa_ref, wb_ref, te_ref,
                war_ref, wbr_ref):
    x = x_ref[...]
    logits = jnp.dot(x, wr_ref[...], preferred_element_type=jnp.float32)
    m = jnp.max(logits, axis=1, keepdims=True)
    ex = jnp.exp(logits - m)
    probs = ex / jnp.sum(ex, axis=1, keepdims=True)

    eio = lax.broadcasted_iota(jnp.int32, (T, NE), 1)
    m1 = jnp.max(probs, axis=1, keepdims=True)
    i1 = jnp.min(jnp.where(probs == m1, eio, NE), axis=1, keepdims=True)
    p2 = jnp.where(eio == i1, -1.0, probs)
    m2 = jnp.max(p2, axis=1, keepdims=True)
    i2 = jnp.min(jnp.where(p2 == m2, eio, NE), axis=1, keepdims=True)
    s = m1 + m2
    wa_ref[...] = m1 / s
    wb_ref[...] = m2 / s
    war_ref[...] = jnp.broadcast_to(m1 / s, (T, 16))
    wbr_ref[...] = jnp.broadcast_to(m2 / s, (T, 16))

    oha = (eio == i1).astype(jnp.float32)
    ohb = (eio == i2).astype(jnp.float32)
    ind = oha + ohb

    pos = jnp.zeros((T, NE), jnp.float32)
    rio = lax.broadcasted_iota(jnp.int32, (T, KB), 0)
    cio = lax.broadcasted_iota(jnp.int32, (T, KB), 1)
    indb = ind.astype(jnp.bfloat16)
    for kb in range(T // KB):
        tri = (rio > cio + kb * KB).astype(jnp.bfloat16)
        pos = pos + jnp.dot(tri, indb[kb * KB:(kb + 1) * KB, :],
                            preferred_element_type=jnp.float32)

    counts = jnp.sum(ind, axis=0, keepdims=True)
    tiles = jnp.floor((counts + (BT - 1)) * (1.0 / BT))
    ii = lax.broadcasted_iota(jnp.int32, (NE, NE), 0)
    jj = lax.broadcasted_iota(jnp.int32, (NE, NE), 1)
    excl = (ii < jj).astype(jnp.float32)
    start_tiles = jnp.dot(tiles, excl, preferred_element_type=jnp.float32)

    dest = start_tiles * BT + pos
    dsta_ref[...] = jnp.sum(dest * oha, axis=1).astype(jnp.int32)
    dstb_ref[...] = jnp.sum(dest * ohb, axis=1).astype(jnp.int32)

    tio = lax.broadcasted_iota(jnp.int32, (NT, NE), 0)
    eio2 = lax.broadcasted_iota(jnp.int32, (NT, NE), 1)
    st_i = start_tiles.astype(jnp.int32)
    nonempty = counts > 0.0
    started = tio >= st_i
    tev = jnp.sum(started.astype(jnp.int32), axis=1) - 1
    run = jnp.sum((started & nonempty).astype(jnp.int32), axis=1) - 1
    par = jnp.bitwise_and(run, 1)
    first = jnp.max(((tio == st_i) & nonempty).astype(jnp.int32), axis=1)
    nxte = jnp.min(jnp.where((tio < st_i) & nonempty, eio2, NE), axis=1)
    hasnx = (nxte < NE).astype(jnp.int32)
    nxte = jnp.minimum(nxte, NE - 1)
    te_ref[...] = jnp.concatenate(
        [tev[None, :], par[None, :], first[None, :],
         nxte[None, :], hasnx[None, :]], axis=0)


_route = pl.pallas_call(
    _route_body,
    out_shape=[
        jax.ShapeDtypeStruct((T,), jnp.int32),
        jax.ShapeDtypeStruct((T,), jnp.int32),
        jax.ShapeDtypeStruct((T, 1), jnp.float32),
        jax.ShapeDtypeStruct((T, 1), jnp.float32),
        jax.ShapeDtypeStruct((5, NT), jnp.int32),
        jax.ShapeDtypeStruct((T, 16), jnp.float32),
        jax.ShapeDtypeStruct((T, 16), jnp.float32),
    ],
)


def _dispatch_body(x_hbm, dsta_hbm, dstb_hbm, xs_hbm, idx_v, rows_v, sem):
    wid = lax.axis_index("s") * NC + lax.axis_index("c")
    base = wid * TPW
    pltpu.sync_copy(x_hbm.at[pl.ds(base, TPW)], rows_v)
    pltpu.sync_copy(dsta_hbm.at[wid], idx_v)
    pltpu.async_copy(rows_v, xs_hbm.at[idx_v], sem).wait()
    pltpu.sync_copy(dstb_hbm.at[wid], idx_v)
    pltpu.async_copy(rows_v, xs_hbm.at[idx_v], sem).wait()


def _gmm_body(meta_ref, xs_ref, w1_hbm, w2_hbm, out_ref, w1s, w2s, sems):
    j = pl.program_id(0)
    e = meta_ref[0, j]
    par = meta_ref[1, j]
    first = meta_ref[2, j]
    nxte = meta_ref[3, j]
    hasnx = meta_ref[4, j]

    @pl.when(j == 0)
    def _start_first_run():
        pltpu.make_async_copy(w1_hbm.at[e], w1s.at[0], sems.at[0]).start()
        pltpu.make_async_copy(w2_hbm.at[e], w2s.at[0], sems.at[0]).start()

    @pl.when(first == 1)
    def _run_boundary():
        pltpu.make_async_copy(w1_hbm.at[e], w1s.at[par], sems.at[par]).wait()
        pltpu.make_async_copy(w2_hbm.at[e], w2s.at[par], sems.at[par]).wait()

        @pl.when(hasnx == 1)
        def _prefetch_next_run():
            pltpu.make_async_copy(
                w1_hbm.at[nxte], w1s.at[1 - par], sems.at[1 - par]).start()
            pltpu.make_async_copy(
                w2_hbm.at[nxte], w2s.at[1 - par], sems.at[1 - par]).start()

    h = jnp.dot(xs_ref[...], w1s[par], preferred_element_type=jnp.float32)
    h = h * (1.0 / (1.0 + jnp.exp(-h)))
    out_ref[...] = jnp.dot(h, w2s[par], preferred_element_type=jnp.float32)


_gmm = pl.pallas_call(
    _gmm_body,
    grid_spec=pltpu.PrefetchScalarGridSpec(
        num_scalar_prefetch=1,
        grid=(NT,),
        in_specs=[
            pl.BlockSpec((BT, DM), lambda j, meta: (j, 0)),
            pl.BlockSpec(memory_space=pl.ANY),
            pl.BlockSpec(memory_space=pl.ANY),
        ],
        out_specs=pl.BlockSpec((BT, DM), lambda j, meta: (j, 0)),
        scratch_shapes=[
            pltpu.VMEM((2, DM, DF), jnp.float32),
            pltpu.VMEM((2, DF, DM), jnp.float32),
            pltpu.SemaphoreType.DMA((2,)),
        ],
    ),
    out_shape=jax.ShapeDtypeStruct((P, DM), jnp.float32),
)


CH = 32


def _combine_sc_body(h2_hbm, dsta_hbm, dstb_hbm, wa_hbm, wb_hbm, out_hbm,
                     ida_v, idb_v, wa_v, wb_v, bufa, bufb, sem):
    wid = lax.axis_index("s") * NC + lax.axis_index("c")
    base = wid * TPW
    pltpu.sync_copy(dsta_hbm.at[wid], ida_v)
    pltpu.sync_copy(dstb_hbm.at[wid], idb_v)
    pltpu.sync_copy(wa_hbm.at[wid], wa_v)
    pltpu.sync_copy(wb_hbm.at[wid], wb_v)
    for c in range(TPW // CH):
        pltpu.async_copy(
            h2_hbm.at[ida_v.at[pl.ds(c * CH, CH)]], bufa, sem).wait()
        pltpu.async_copy(
            h2_hbm.at[idb_v.at[pl.ds(c * CH, CH)]], bufb, sem).wait()

        def row_body(r, _, c=c):
            tok = c * CH + r
            wa_s = wa_v[tok]
            wb_s = wb_v[tok]

            def col_body(q, _):
                a = bufa[r, pl.ds(q * 16, 16)]
                b = bufb[r, pl.ds(q * 16, 16)]
                bufa[r, pl.ds(q * 16, 16)] = wa_s * a + wb_s * b
                return 0

            lax.fori_loop(0, DM // 16, col_body, 0, unroll=4)
            return 0

        lax.fori_loop(0, CH, row_body, 0)
        pltpu.sync_copy(bufa, out_hbm.at[pl.ds(base + c * CH, CH)])


@functools.lru_cache(maxsize=1)
def _sc_kernels():
    mesh = plsc.VectorSubcoreMesh(
        core_axis_name="c", subcore_axis_name="s",
        num_cores=NC, num_subcores=NS)
    dispatch = pl.kernel(
        _dispatch_body,
        mesh=mesh,
        out_type=jax.ShapeDtypeStruct((P, DM), jnp.float32),
        scratch_types=[
            pltpu.VMEM((TPW,), jnp.int32),
            pltpu.VMEM((TPW, DM), jnp.float32),
            pltpu.SemaphoreType.DMA,
        ],
    )
    combine_sc = pl.kernel(
        _combine_sc_body,
        mesh=mesh,
        out_type=jax.ShapeDtypeStruct((T, DM), jnp.float32),
        scratch_types=[
            pltpu.VMEM((TPW,), jnp.int32),
            pltpu.VMEM((TPW,), jnp.int32),
            pltpu.VMEM((TPW, 16), jnp.float32),
            pltpu.VMEM((TPW, 16), jnp.float32),
            pltpu.VMEM((CH, DM), jnp.float32),
            pltpu.VMEM((CH, DM), jnp.float32),
            pltpu.SemaphoreType.DMA,
        ],
    )
    return dispatch, combine_sc


def kernel(x, w_router, w1, w2):
    dispatch, combine_sc = _sc_kernels()
    dsta, dstb, wa, wb, meta, war, wbr = _route(x, w_router)
    da = dsta.reshape(NW, TPW)
    db = dstb.reshape(NW, TPW)
    xs = dispatch(x, da, db)
    h2 = _gmm(meta, xs, w1, w2)
    return combine_sc(h2, da, db,
                      war.reshape(NW, TPW, 16), wbr.reshape(NW, TPW, 16))


# speedup vs baseline: 1.1407x; 1.0203x over previous
import functools

import jax
import jax.numpy as jnp
from jax import lax
from jax.experimental import pallas as pl
from jax.experimental.pallas import tpu as pltpu
from jax.experimental.pallas import tpu_sc as plsc

NE = 8
DM = 1024
DF = 2048
T = 2048
BT = 128
NT = (T * 2 + NE * (BT - 1) + BT - 1) // BT
P = NT * BT
NC, NS = 2, 16
NW = NC * NS
TPW = T // NW
KB = 256


def _route_body(x_ref, wr_ref, dsta_ref, dstb_ref, wa_ref, wb_ref, te_ref,
                war_ref, wbr_ref):
    x = x_ref[...]
    logits = jnp.dot(x, wr_ref[...], preferred_element_type=jnp.float32)
    m = jnp.max(logits, axis=1, keepdims=True)
    ex = jnp.exp(logits - m)
    probs = ex / jnp.sum(ex, axis=1, keepdims=True)

    eio = lax.broadcasted_iota(jnp.int32, (T, NE), 1)
    m1 = jnp.max(probs, axis=1, keepdims=True)
    i1 = jnp.min(jnp.where(probs == m1, eio, NE), axis=1, keepdims=True)
    p2 = jnp.where(eio == i1, -1.0, probs)
    m2 = jnp.max(p2, axis=1, keepdims=True)
    i2 = jnp.min(jnp.where(p2 == m2, eio, NE), axis=1, keepdims=True)
    s = m1 + m2
    wa_ref[...] = m1 / s
    wb_ref[...] = m2 / s
    war_ref[...] = jnp.broadcast_to(m1 / s, (T, 16))
    wbr_ref[...] = jnp.broadcast_to(m2 / s, (T, 16))

    oha = (eio == i1).astype(jnp.float32)
    ohb = (eio == i2).astype(jnp.float32)
    ind = oha + ohb

    pos = jnp.zeros((T, NE), jnp.float32)
    rio = lax.broadcasted_iota(jnp.int32, (T, KB), 0)
    cio = lax.broadcasted_iota(jnp.int32, (T, KB), 1)
    indb = ind.astype(jnp.bfloat16)
    for kb in range(T // KB):
        tri = (rio > cio + kb * KB).astype(jnp.bfloat16)
        pos = pos + jnp.dot(tri, indb[kb * KB:(kb + 1) * KB, :],
                            preferred_element_type=jnp.float32)

    counts = jnp.sum(ind, axis=0, keepdims=True)
    tiles = jnp.floor((counts + (BT - 1)) * (1.0 / BT))
    ii = lax.broadcasted_iota(jnp.int32, (NE, NE), 0)
    jj = lax.broadcasted_iota(jnp.int32, (NE, NE), 1)
    excl = (ii < jj).astype(jnp.float32)
    start_tiles = jnp.dot(tiles, excl, preferred_element_type=jnp.float32)

    dest = start_tiles * BT + pos
    dsta_ref[...] = jnp.sum(dest * oha, axis=1).astype(jnp.int32)
    dstb_ref[...] = jnp.sum(dest * ohb, axis=1).astype(jnp.int32)

    tio = lax.broadcasted_iota(jnp.int32, (NT, NE), 0)
    eio2 = lax.broadcasted_iota(jnp.int32, (NT, NE), 1)
    st_i = start_tiles.astype(jnp.int32)
    nonempty = counts > 0.0
    started = tio >= st_i
    tev = jnp.sum(started.astype(jnp.int32), axis=1) - 1
    run = jnp.sum((started & nonempty).astype(jnp.int32), axis=1) - 1
    par = jnp.bitwise_and(run, 1)
    first = jnp.max(((tio == st_i) & nonempty).astype(jnp.int32), axis=1)
    nxte = jnp.min(jnp.where((tio < st_i) & nonempty, eio2, NE), axis=1)
    hasnx = (nxte < NE).astype(jnp.int32)
    nxte = jnp.minimum(nxte, NE - 1)
    totv = jnp.sum(tiles, axis=1, keepdims=True)
    real = jnp.sum((tio[:, :1] < totv).astype(jnp.int32), axis=1)
    te_ref[...] = jnp.concatenate(
        [tev[None, :], par[None, :], first[None, :],
         nxte[None, :], hasnx[None, :], real[None, :]], axis=0)


_route = pl.pallas_call(
    _route_body,
    out_shape=[
        jax.ShapeDtypeStruct((T,), jnp.int32),
        jax.ShapeDtypeStruct((T,), jnp.int32),
        jax.ShapeDtypeStruct((T, 1), jnp.float32),
        jax.ShapeDtypeStruct((T, 1), jnp.float32),
        jax.ShapeDtypeStruct((6, NT), jnp.int32),
        jax.ShapeDtypeStruct((T, 16), jnp.float32),
        jax.ShapeDtypeStruct((T, 16), jnp.float32),
    ],
)


def _dispatch_body(x_hbm, dsta_hbm, dstb_hbm, xs_hbm, idx_v, rows_v, sem):
    wid = lax.axis_index("s") * NC + lax.axis_index("c")
    base = wid * TPW
    pltpu.sync_copy(x_hbm.at[pl.ds(base, TPW)], rows_v)
    pltpu.sync_copy(dsta_hbm.at[wid], idx_v)
    pltpu.async_copy(rows_v, xs_hbm.at[idx_v], sem).wait()
    pltpu.sync_copy(dstb_hbm.at[wid], idx_v)
    pltpu.async_copy(rows_v, xs_hbm.at[idx_v], sem).wait()


def _gmm_body(meta_ref, xs_ref, w1_hbm, w2_hbm, out_ref, w1s, w2s, sems):
    j = pl.program_id(0)
    e = meta_ref[0, j]
    par = meta_ref[1, j]
    first = meta_ref[2, j]
    nxte = meta_ref[3, j]
    hasnx = meta_ref[4, j]

    @pl.when(j == 0)
    def _start_first_run():
        pltpu.make_async_copy(w1_hbm.at[e], w1s.at[0], sems.at[0]).start()
        pltpu.make_async_copy(w2_hbm.at[e], w2s.at[0], sems.at[0]).start()

    @pl.when(first == 1)
    def _run_boundary():
        pltpu.make_async_copy(w1_hbm.at[e], w1s.at[par], sems.at[par]).wait()
        pltpu.make_async_copy(w2_hbm.at[e], w2s.at[par], sems.at[par]).wait()

        @pl.when(hasnx == 1)
        def _prefetch_next_run():
            pltpu.make_async_copy(
                w1_hbm.at[nxte], w1s.at[1 - par], sems.at[1 - par]).start()
            pltpu.make_async_copy(
                w2_hbm.at[nxte], w2s.at[1 - par], sems.at[1 - par]).start()

    @pl.when(meta_ref[5, j] == 1)
    def _compute():
        h = jnp.dot(xs_ref[...], w1s[par],
                    preferred_element_type=jnp.float32)
        h = h * (1.0 / (1.0 + jnp.exp(-h)))
        out_ref[...] = jnp.dot(h, w2s[par],
                               preferred_element_type=jnp.float32)


_gmm = pl.pallas_call(
    _gmm_body,
    grid_spec=pltpu.PrefetchScalarGridSpec(
        num_scalar_prefetch=1,
        grid=(NT,),
        in_specs=[
            pl.BlockSpec((BT, DM), lambda j, meta: (j, 0)),
            pl.BlockSpec(memory_space=pl.ANY),
            pl.BlockSpec(memory_space=pl.ANY),
        ],
        out_specs=pl.BlockSpec((BT, DM), lambda j, meta: (j, 0)),
        scratch_shapes=[
            pltpu.VMEM((2, DM, DF), jnp.float32),
            pltpu.VMEM((2, DF, DM), jnp.float32),
            pltpu.SemaphoreType.DMA((2,)),
        ],
    ),
    out_shape=jax.ShapeDtypeStruct((P, DM), jnp.float32),
)


CH = 32


def _combine_sc_body(h2_hbm, dsta_hbm, dstb_hbm, wa_hbm, wb_hbm, out_hbm,
                     ida_v, idb_v, wa_v, wb_v, bufa, bufb, sem):
    wid = lax.axis_index("s") * NC + lax.axis_index("c")
    base = wid * TPW
    pltpu.sync_copy(dsta_hbm.at[wid], ida_v)
    pltpu.sync_copy(dstb_hbm.at[wid], idb_v)
    pltpu.sync_copy(wa_hbm.at[wid], wa_v)
    pltpu.sync_copy(wb_hbm.at[wid], wb_v)
    for c in range(TPW // CH):
        pltpu.async_copy(
            h2_hbm.at[ida_v.at[pl.ds(c * CH, CH)]], bufa, sem).wait()
        pltpu.async_copy(
            h2_hbm.at[idb_v.at[pl.ds(c * CH, CH)]], bufb, sem).wait()

        def row_body(r, _, c=c):
            tok = c * CH + r
            wa_s = wa_v[tok]
            wb_s = wb_v[tok]

            def col_body(q, _):
                a = bufa[r, pl.ds(q * 16, 16)]
                b = bufb[r, pl.ds(q * 16, 16)]
                bufa[r, pl.ds(q * 16, 16)] = wa_s * a + wb_s * b
                return 0

            lax.fori_loop(0, DM // 16, col_body, 0, unroll=4)
            return 0

        lax.fori_loop(0, CH, row_body, 0)
        pltpu.sync_copy(bufa, out_hbm.at[pl.ds(base + c * CH, CH)])


@functools.lru_cache(maxsize=1)
def _sc_kernels():
    mesh = plsc.VectorSubcoreMesh(
        core_axis_name="c", subcore_axis_name="s",
        num_cores=NC, num_subcores=NS)
    dispatch = pl.kernel(
        _dispatch_body,
        mesh=mesh,
        out_type=jax.ShapeDtypeStruct((P, DM), jnp.float32),
        scratch_types=[
            pltpu.VMEM((TPW,), jnp.int32),
            pltpu.VMEM((TPW, DM), jnp.float32),
            pltpu.SemaphoreType.DMA,
        ],
    )
    combine_sc = pl.kernel(
        _combine_sc_body,
        mesh=mesh,
        out_type=jax.ShapeDtypeStruct((T, DM), jnp.float32),
        scratch_types=[
            pltpu.VMEM((TPW,), jnp.int32),
            pltpu.VMEM((TPW,), jnp.int32),
            pltpu.VMEM((TPW, 16), jnp.float32),
            pltpu.VMEM((TPW, 16), jnp.float32),
            pltpu.VMEM((CH, DM), jnp.float32),
            pltpu.VMEM((CH, DM), jnp.float32),
            pltpu.SemaphoreType.DMA,
        ],
    )
    return dispatch, combine_sc


def kernel(x, w_router, w1, w2):
    dispatch, combine_sc = _sc_kernels()
    dsta, dstb, wa, wb, meta, war, wbr = _route(x, w_router)
    da = dsta.reshape(NW, TPW)
    db = dstb.reshape(NW, TPW)
    xs = dispatch(x, da, db)
    h2 = _gmm(meta, xs, w1, w2)
    return combine_sc(h2, da, db,
                      war.reshape(NW, TPW, 16), wbr.reshape(NW, TPW, 16))


# speedup vs baseline: 1.1543x; 1.0120x over previous
import functools

import jax
import jax.numpy as jnp
from jax import lax
from jax.experimental import pallas as pl
from jax.experimental.pallas import tpu as pltpu
from jax.experimental.pallas import tpu_sc as plsc

NE = 8
DM = 1024
DF = 2048
T = 2048
BT = 128
NT = (T * 2 + NE * (BT - 1) + BT - 1) // BT
P = NT * BT
NC, NS = 2, 16
NW = NC * NS
TPW = T // NW
KB = 256


def _route_body(x_ref, wr_ref, dsta_ref, dstb_ref, wa_ref, wb_ref, te_ref,
                war_ref, wbr_ref):
    x = x_ref[...]
    logits = jnp.dot(x, wr_ref[...], preferred_element_type=jnp.float32)
    m = jnp.max(logits, axis=1, keepdims=True)
    ex = jnp.exp(logits - m)
    probs = ex / jnp.sum(ex, axis=1, keepdims=True)

    eio = lax.broadcasted_iota(jnp.int32, (T, NE), 1)
    m1 = jnp.max(probs, axis=1, keepdims=True)
    i1 = jnp.min(jnp.where(probs == m1, eio, NE), axis=1, keepdims=True)
    p2 = jnp.where(eio == i1, -1.0, probs)
    m2 = jnp.max(p2, axis=1, keepdims=True)
    i2 = jnp.min(jnp.where(p2 == m2, eio, NE), axis=1, keepdims=True)
    s = m1 + m2
    wa_ref[...] = m1 / s
    wb_ref[...] = m2 / s
    war_ref[...] = jnp.broadcast_to(m1 / s, (T, 16))
    wbr_ref[...] = jnp.broadcast_to(m2 / s, (T, 16))

    oha = (eio == i1).astype(jnp.float32)
    ohb = (eio == i2).astype(jnp.float32)
    ind = oha + ohb

    pos = jnp.zeros((T, NE), jnp.float32)
    rio = lax.broadcasted_iota(jnp.int32, (T, KB), 0)
    cio = lax.broadcasted_iota(jnp.int32, (T, KB), 1)
    indb = ind.astype(jnp.bfloat16)
    for kb in range(T // KB):
        tri = (rio > cio + kb * KB).astype(jnp.bfloat16)
        pos = pos + jnp.dot(tri, indb[kb * KB:(kb + 1) * KB, :],
                            preferred_element_type=jnp.float32)

    counts = jnp.sum(ind, axis=0, keepdims=True)
    tiles = jnp.floor((counts + (BT - 1)) * (1.0 / BT))
    ii = lax.broadcasted_iota(jnp.int32, (NE, NE), 0)
    jj = lax.broadcasted_iota(jnp.int32, (NE, NE), 1)
    excl = (ii < jj).astype(jnp.float32)
    start_tiles = jnp.dot(tiles, excl, preferred_element_type=jnp.float32)

    dest = start_tiles * BT + pos
    dsta_ref[...] = jnp.sum(dest * oha, axis=1).astype(jnp.int32)
    dstb_ref[...] = jnp.sum(dest * ohb, axis=1).astype(jnp.int32)

    tio = lax.broadcasted_iota(jnp.int32, (NT, NE), 0)
    eio2 = lax.broadcasted_iota(jnp.int32, (NT, NE), 1)
    st_i = start_tiles.astype(jnp.int32)
    nonempty = counts > 0.0
    started = tio >= st_i
    tev = jnp.sum(started.astype(jnp.int32), axis=1) - 1
    run = jnp.sum((started & nonempty).astype(jnp.int32), axis=1) - 1
    par = jnp.bitwise_and(run, 1)
    first = jnp.max(((tio == st_i) & nonempty).astype(jnp.int32), axis=1)
    nxte = jnp.min(jnp.where((tio < st_i) & nonempty, eio2, NE), axis=1)
    hasnx = (nxte < NE).astype(jnp.int32)
    nxte = jnp.minimum(nxte, NE - 1)
    totv = jnp.sum(tiles, axis=1, keepdims=True)
    real = jnp.sum((tio[:, :1] < totv).astype(jnp.int32), axis=1)
    te_ref[...] = jnp.concatenate(
        [tev[None, :], par[None, :], first[None, :],
         nxte[None, :], hasnx[None, :], real[None, :]], axis=0)


_route = pl.pallas_call(
    _route_body,
    out_shape=[
        jax.ShapeDtypeStruct((T,), jnp.int32),
        jax.ShapeDtypeStruct((T,), jnp.int32),
        jax.ShapeDtypeStruct((T, 1), jnp.float32),
        jax.ShapeDtypeStruct((T, 1), jnp.float32),
        jax.ShapeDtypeStruct((6, NT), jnp.int32),
        jax.ShapeDtypeStruct((T, 16), jnp.float32),
        jax.ShapeDtypeStruct((T, 16), jnp.float32),
    ],
)


def _dispatch_body(x_hbm, dsta_hbm, dstb_hbm, xs_hbm, ida_v, idb_v, rows_v,
                   sem):
    wid = lax.axis_index("s") * NC + lax.axis_index("c")
    base = wid * TPW
    pltpu.sync_copy(x_hbm.at[pl.ds(base, TPW)], rows_v)
    pltpu.sync_copy(dsta_hbm.at[wid], ida_v)
    pltpu.sync_copy(dstb_hbm.at[wid], idb_v)
    cpa = pltpu.async_copy(rows_v, xs_hbm.at[ida_v], sem)
    cpb = pltpu.async_copy(rows_v, xs_hbm.at[idb_v], sem)
    cpa.wait()
    cpb.wait()


def _gmm_body(meta_ref, xs_ref, w1_hbm, w2_hbm, out_ref, w1s, w2s, sems):
    j = pl.program_id(0)
    e = meta_ref[0, j]
    par = meta_ref[1, j]
    first = meta_ref[2, j]
    nxte = meta_ref[3, j]
    hasnx = meta_ref[4, j]

    @pl.when(j == 0)
    def _start_first_run():
        pltpu.make_async_copy(w1_hbm.at[e], w1s.at[0], sems.at[0]).start()
        pltpu.make_async_copy(w2_hbm.at[e], w2s.at[0], sems.at[0]).start()

    @pl.when(first == 1)
    def _run_boundary():
        pltpu.make_async_copy(w1_hbm.at[e], w1s.at[par], sems.at[par]).wait()
        pltpu.make_async_copy(w2_hbm.at[e], w2s.at[par], sems.at[par]).wait()

        @pl.when(hasnx == 1)
        def _prefetch_next_run():
            pltpu.make_async_copy(
                w1_hbm.at[nxte], w1s.at[1 - par], sems.at[1 - par]).start()
            pltpu.make_async_copy(
                w2_hbm.at[nxte], w2s.at[1 - par], sems.at[1 - par]).start()

    @pl.when(meta_ref[5, j] == 1)
    def _compute():
        h = jnp.dot(xs_ref[...], w1s[par],
                    preferred_element_type=jnp.float32)
        h = h * (1.0 / (1.0 + jnp.exp(-h)))
        out_ref[...] = jnp.dot(h, w2s[par],
                               preferred_element_type=jnp.float32)


_gmm = pl.pallas_call(
    _gmm_body,
    grid_spec=pltpu.PrefetchScalarGridSpec(
        num_scalar_prefetch=1,
        grid=(NT,),
        in_specs=[
            pl.BlockSpec((BT, DM), lambda j, meta: (j, 0)),
            pl.BlockSpec(memory_space=pl.ANY),
            pl.BlockSpec(memory_space=pl.ANY),
        ],
        out_specs=pl.BlockSpec((BT, DM), lambda j, meta: (j, 0)),
        scratch_shapes=[
            pltpu.VMEM((2, DM, DF), jnp.float32),
            pltpu.VMEM((2, DF, DM), jnp.float32),
            pltpu.SemaphoreType.DMA((2,)),
        ],
    ),
    out_shape=jax.ShapeDtypeStruct((P, DM), jnp.float32),
)


CH = 32


def _combine_sc_body(h2_hbm, dsta_hbm, dstb_hbm, wa_hbm, wb_hbm, out_hbm,
                     ida_v, idb_v, wa_v, wb_v, bufa, bufb, sem):
    wid = lax.axis_index("s") * NC + lax.axis_index("c")
    base = wid * TPW
    pltpu.sync_copy(dsta_hbm.at[wid], ida_v)
    pltpu.sync_copy(dstb_hbm.at[wid], idb_v)
    pltpu.sync_copy(wa_hbm.at[wid], wa_v)
    pltpu.sync_copy(wb_hbm.at[wid], wb_v)
    for c in range(TPW // CH):
        cpa = pltpu.async_copy(
            h2_hbm.at[ida_v.at[pl.ds(c * CH, CH)]], bufa, sem)
        cpb = pltpu.async_copy(
            h2_hbm.at[idb_v.at[pl.ds(c * CH, CH)]], bufb, sem)
        cpa.wait()
        cpb.wait()

        def row_body(r, _, c=c):
            tok = c * CH + r
            wa_s = wa_v[tok]
            wb_s = wb_v[tok]

            def col_body(q, _):
                a = bufa[r, pl.ds(q * 16, 16)]
                b = bufb[r, pl.ds(q * 16, 16)]
                bufa[r, pl.ds(q * 16, 16)] = wa_s * a + wb_s * b
                return 0

            lax.fori_loop(0, DM // 16, col_body, 0, unroll=4)
            return 0

        lax.fori_loop(0, CH, row_body, 0)
        pltpu.sync_copy(bufa, out_hbm.at[pl.ds(base + c * CH, CH)])


@functools.lru_cache(maxsize=1)
def _sc_kernels():
    mesh = plsc.VectorSubcoreMesh(
        core_axis_name="c", subcore_axis_name="s",
        num_cores=NC, num_subcores=NS)
    dispatch = pl.kernel(
        _dispatch_body,
        mesh=mesh,
        out_type=jax.ShapeDtypeStruct((P, DM), jnp.float32),
        scratch_types=[
            pltpu.VMEM((TPW,), jnp.int32),
            pltpu.VMEM((TPW,), jnp.int32),
            pltpu.VMEM((TPW, DM), jnp.float32),
            pltpu.SemaphoreType.DMA,
        ],
    )
    combine_sc = pl.kernel(
        _combine_sc_body,
        mesh=mesh,
        out_type=jax.ShapeDtypeStruct((T, DM), jnp.float32),
        scratch_types=[
            pltpu.VMEM((TPW,), jnp.int32),
            pltpu.VMEM((TPW,), jnp.int32),
            pltpu.VMEM((TPW, 16), jnp.float32),
            pltpu.VMEM((TPW, 16), jnp.float32),
            pltpu.VMEM((CH, DM), jnp.float32),
            pltpu.VMEM((CH, DM), jnp.float32),
            pltpu.SemaphoreType.DMA,
        ],
    )
    return dispatch, combine_sc


def kernel(x, w_router, w1, w2):
    dispatch, combine_sc = _sc_kernels()
    dsta, dstb, wa, wb, meta, war, wbr = _route(x, w_router)
    da = dsta.reshape(NW, TPW)
    db = dstb.reshape(NW, TPW)
    xs = dispatch(x, da, db)
    h2 = _gmm(meta, xs, w1, w2)
    return combine_sc(h2, da, db,
                      war.reshape(NW, TPW, 16), wbr.reshape(NW, TPW, 16))


# speedup vs baseline: 1.1746x; 1.0176x over previous
import functools

import jax
import jax.numpy as jnp
from jax import lax
from jax.experimental import pallas as pl
from jax.experimental.pallas import tpu as pltpu
from jax.experimental.pallas import tpu_sc as plsc

NE = 8
DM = 1024
DF = 2048
T = 2048
BT = 128
NT = (T * 2 + NE * (BT - 1) + BT - 1) // BT
P = NT * BT
NC, NS = 2, 16
NW = NC * NS
TPW = T // NW
KB = 256


def _route_body(x_ref, wr_ref, dsta_ref, dstb_ref, wa_ref, wb_ref, te_ref,
                war_ref, wbr_ref):
    x = x_ref[...]
    logits = jnp.dot(x, wr_ref[...], preferred_element_type=jnp.float32)
    m = jnp.max(logits, axis=1, keepdims=True)
    ex = jnp.exp(logits - m)
    probs = ex / jnp.sum(ex, axis=1, keepdims=True)

    eio = lax.broadcasted_iota(jnp.int32, (T, NE), 1)
    m1 = jnp.max(probs, axis=1, keepdims=True)
    i1 = jnp.min(jnp.where(probs == m1, eio, NE), axis=1, keepdims=True)
    p2 = jnp.where(eio == i1, -1.0, probs)
    m2 = jnp.max(p2, axis=1, keepdims=True)
    i2 = jnp.min(jnp.where(p2 == m2, eio, NE), axis=1, keepdims=True)
    s = m1 + m2
    wa_ref[...] = m1 / s
    wb_ref[...] = m2 / s
    war_ref[...] = jnp.broadcast_to(m1 / s, (T, 16))
    wbr_ref[...] = jnp.broadcast_to(m2 / s, (T, 16))

    oha = (eio == i1).astype(jnp.float32)
    ohb = (eio == i2).astype(jnp.float32)
    ind = oha + ohb

    pos = jnp.zeros((T, NE), jnp.float32)
    rio = lax.broadcasted_iota(jnp.int32, (T, KB), 0)
    cio = lax.broadcasted_iota(jnp.int32, (T, KB), 1)
    indb = ind.astype(jnp.bfloat16)
    for kb in range(T // KB):
        tri = (rio > cio + kb * KB).astype(jnp.bfloat16)
        pos = pos + jnp.dot(tri, indb[kb * KB:(kb + 1) * KB, :],
                            preferred_element_type=jnp.float32)

    counts = jnp.sum(ind, axis=0, keepdims=True)
    tiles = jnp.floor((counts + (BT - 1)) * (1.0 / BT))
    ii = lax.broadcasted_iota(jnp.int32, (NE, NE), 0)
    jj = lax.broadcasted_iota(jnp.int32, (NE, NE), 1)
    excl = (ii < jj).astype(jnp.float32)
    start_tiles = jnp.dot(tiles, excl, preferred_element_type=jnp.float32)

    dest = start_tiles * BT + pos
    dsta_ref[...] = jnp.sum(dest * oha, axis=1).astype(jnp.int32)
    dstb_ref[...] = jnp.sum(dest * ohb, axis=1).astype(jnp.int32)

    tio = lax.broadcasted_iota(jnp.int32, (NT, NE), 0)
    eio2 = lax.broadcasted_iota(jnp.int32, (NT, NE), 1)
    st_i = start_tiles.astype(jnp.int32)
    nonempty = counts > 0.0
    started = tio >= st_i
    tev = jnp.sum(started.astype(jnp.int32), axis=1) - 1
    run = jnp.sum((started & nonempty).astype(jnp.int32), axis=1) - 1
    par = jnp.bitwise_and(run, 1)
    first = jnp.max(((tio == st_i) & nonempty).astype(jnp.int32), axis=1)
    nxte = jnp.min(jnp.where((tio < st_i) & nonempty, eio2, NE), axis=1)
    hasnx = (nxte < NE).astype(jnp.int32)
    nxte = jnp.minimum(nxte, NE - 1)
    totv = jnp.sum(tiles, axis=1, keepdims=True)
    real = jnp.sum((tio[:, :1] < totv).astype(jnp.int32), axis=1)
    te_ref[...] = jnp.concatenate(
        [tev[None, :], par[None, :], first[None, :],
         nxte[None, :], hasnx[None, :], real[None, :]], axis=0)


_route = pl.pallas_call(
    _route_body,
    out_shape=[
        jax.ShapeDtypeStruct((T,), jnp.int32),
        jax.ShapeDtypeStruct((T,), jnp.int32),
        jax.ShapeDtypeStruct((T, 1), jnp.float32),
        jax.ShapeDtypeStruct((T, 1), jnp.float32),
        jax.ShapeDtypeStruct((6, NT), jnp.int32),
        jax.ShapeDtypeStruct((T, 16), jnp.float32),
        jax.ShapeDtypeStruct((T, 16), jnp.float32),
    ],
)


def _dispatch_body(x_hbm, dsta_hbm, dstb_hbm, xs_hbm, ida_v, idb_v, rows_v,
                   sem):
    wid = lax.axis_index("s") * NC + lax.axis_index("c")
    base = wid * TPW
    pltpu.sync_copy(x_hbm.at[pl.ds(base, TPW)], rows_v)
    pltpu.sync_copy(dsta_hbm.at[wid], ida_v)
    pltpu.sync_copy(dstb_hbm.at[wid], idb_v)
    cpa = pltpu.async_copy(rows_v, xs_hbm.at[ida_v], sem)
    cpb = pltpu.async_copy(rows_v, xs_hbm.at[idb_v], sem)
    cpa.wait()
    cpb.wait()


def _gmm_body(meta_ref, xs_ref, w1_hbm, w2_hbm, out_ref, w1s, w2s, sems):
    j = pl.program_id(0)
    e = meta_ref[0, j]
    par = meta_ref[1, j]
    first = meta_ref[2, j]
    nxte = meta_ref[3, j]
    hasnx = meta_ref[4, j]

    @pl.when(j == 0)
    def _start_first_run():
        pltpu.make_async_copy(w1_hbm.at[e], w1s.at[0], sems.at[0]).start()
        pltpu.make_async_copy(w2_hbm.at[e], w2s.at[0], sems.at[0]).start()

    @pl.when(first == 1)
    def _run_boundary():
        pltpu.make_async_copy(w1_hbm.at[e], w1s.at[par], sems.at[par]).wait()
        pltpu.make_async_copy(w2_hbm.at[e], w2s.at[par], sems.at[par]).wait()

        @pl.when(hasnx == 1)
        def _prefetch_next_run():
            pltpu.make_async_copy(
                w1_hbm.at[nxte], w1s.at[1 - par], sems.at[1 - par]).start()
            pltpu.make_async_copy(
                w2_hbm.at[nxte], w2s.at[1 - par], sems.at[1 - par]).start()

    @pl.when(meta_ref[5, j] == 1)
    def _compute():
        h = jnp.dot(xs_ref[...], w1s[par],
                    preferred_element_type=jnp.float32)
        h = h * (1.0 / (1.0 + jnp.exp(-h)))
        out_ref[...] = jnp.dot(h, w2s[par],
                               preferred_element_type=jnp.float32)


_gmm = pl.pallas_call(
    _gmm_body,
    grid_spec=pltpu.PrefetchScalarGridSpec(
        num_scalar_prefetch=1,
        grid=(NT,),
        in_specs=[
            pl.BlockSpec((BT, DM), lambda j, meta: (j, 0)),
            pl.BlockSpec(memory_space=pl.ANY),
            pl.BlockSpec(memory_space=pl.ANY),
        ],
        out_specs=pl.BlockSpec((BT, DM), lambda j, meta: (j, 0)),
        scratch_shapes=[
            pltpu.VMEM((2, DM, DF), jnp.float32),
            pltpu.VMEM((2, DF, DM), jnp.float32),
            pltpu.SemaphoreType.DMA((2,)),
        ],
    ),
    out_shape=jax.ShapeDtypeStruct((P, DM), jnp.float32),
)


CH = 16
NCH = TPW // CH


def _combine_sc_body(h2_hbm, dsta_hbm, dstb_hbm, wa_hbm, wb_hbm, out_hbm,
                     ida_v, idb_v, wa_v, wb_v, bufa, bufb, sems):
    wid = lax.axis_index("s") * NC + lax.axis_index("c")
    base = wid * TPW
    pltpu.sync_copy(dsta_hbm.at[wid], ida_v)
    pltpu.sync_copy(dstb_hbm.at[wid], idb_v)
    pltpu.sync_copy(wa_hbm.at[wid], wa_v)
    pltpu.sync_copy(wb_hbm.at[wid], wb_v)

    def fire(c, sl):
        cpa = pltpu.async_copy(
            h2_hbm.at[ida_v.at[pl.ds(c * CH, CH)]], bufa.at[sl], sems.at[sl])
        cpb = pltpu.async_copy(
            h2_hbm.at[idb_v.at[pl.ds(c * CH, CH)]], bufb.at[sl], sems.at[sl])
        return cpa, cpb

    pending = fire(0, 0)
    for c in range(NCH):
        sl = c & 1
        cpa, cpb = pending
        if c + 1 < NCH:
            pending = fire(c + 1, 1 - sl)
        cpa.wait()
        cpb.wait()

        def row_body(r, _, c=c, sl=sl):
            tok = c * CH + r
            wa_s = wa_v[tok]
            wb_s = wb_v[tok]

            def col_body(q, _):
                a = bufa[sl, r, pl.ds(q * 16, 16)]
                b = bufb[sl, r, pl.ds(q * 16, 16)]
                bufa[sl, r, pl.ds(q * 16, 16)] = wa_s * a + wb_s * b
                return 0

            lax.fori_loop(0, DM // 16, col_body, 0, unroll=4)
            return 0

        lax.fori_loop(0, CH, row_body, 0)
        pltpu.sync_copy(bufa.at[sl], out_hbm.at[pl.ds(base + c * CH, CH)])


@functools.lru_cache(maxsize=1)
def _sc_kernels():
    mesh = plsc.VectorSubcoreMesh(
        core_axis_name="c", subcore_axis_name="s",
        num_cores=NC, num_subcores=NS)
    dispatch = pl.kernel(
        _dispatch_body,
        mesh=mesh,
        out_type=jax.ShapeDtypeStruct((P, DM), jnp.float32),
        scratch_types=[
            pltpu.VMEM((TPW,), jnp.int32),
            pltpu.VMEM((TPW,), jnp.int32),
            pltpu.VMEM((TPW, DM), jnp.float32),
            pltpu.SemaphoreType.DMA,
        ],
    )
    combine_sc = pl.kernel(
        _combine_sc_body,
        mesh=mesh,
        out_type=jax.ShapeDtypeStruct((T, DM), jnp.float32),
        scratch_types=[
            pltpu.VMEM((TPW,), jnp.int32),
            pltpu.VMEM((TPW,), jnp.int32),
            pltpu.VMEM((TPW, 16), jnp.float32),
            pltpu.VMEM((TPW, 16), jnp.float32),
            pltpu.VMEM((2, CH, DM), jnp.float32),
            pltpu.VMEM((2, CH, DM), jnp.float32),
            pltpu.SemaphoreType.DMA((2,)),
        ],
    )
    return dispatch, combine_sc


def kernel(x, w_router, w1, w2):
    dispatch, combine_sc = _sc_kernels()
    dsta, dstb, wa, wb, meta, war, wbr = _route(x, w_router)
    da = dsta.reshape(NW, TPW)
    db = dstb.reshape(NW, TPW)
    xs = dispatch(x, da, db)
    h2 = _gmm(meta, xs, w1, w2)
    return combine_sc(h2, da, db,
                      war.reshape(NW, TPW, 16), wbr.reshape(NW, TPW, 16))
